# initial kernel scaffold (unmeasured)
import jax
import jax.numpy as jnp
from jax import lax
from jax.experimental import pallas as pl
from jax.experimental.pallas import tpu as pltpu


def kernel(
    x,
):
    def body(*refs):
        pass

    out_shape = jax.ShapeDtypeStruct(..., jnp.float32)
    return pl.pallas_call(body, out_shape=out_shape)(...)


# baseline (device time: 18977 ns/iter reference)
import jax
import jax.numpy as jnp
from jax import lax
from jax.experimental import pallas as pl
from jax.experimental.pallas import tpu as pltpu

N_DEV = 8


def kernel(x):
    _, m, n = x.shape
    ch = m // N_DEV

    def body(x_ref, out_ref, xb_ref, comm_ref, red_ref,
             send1, recv1, send2, recv2):
        me = lax.axis_index("i")

        xb_ref[...] = x_ref[0].astype(jnp.bfloat16)

        p1_sends = []
        for q in range(N_DEV):
            rdma = pltpu.make_async_remote_copy(
                src_ref=xb_ref.at[pl.ds(q * ch, ch), :],
                dst_ref=comm_ref.at[me],
                send_sem=send1.at[q],
                recv_sem=recv1.at[me],
                device_id=(q,),
                device_id_type=pl.DeviceIdType.MESH,
            )
            p1_sends.append(rdma)

            @pl.when(me != q)
            def _(rdma=rdma):
                rdma.start()

        comm_ref[pl.ds(me, 1)] = xb_ref[pl.ds(me * ch, ch), :][None]

        for p in range(N_DEV):
            recv = pltpu.make_async_remote_copy(
                src_ref=comm_ref.at[p],
                dst_ref=comm_ref.at[p],
                send_sem=send1.at[p],
                recv_sem=recv1.at[p],
                device_id=(p,),
                device_id_type=pl.DeviceIdType.MESH,
            )

            @pl.when(me != p)
            def _(recv=recv):
                recv.wait_recv()

        red = comm_ref[0]
        for p in range(1, N_DEV):
            red = red + comm_ref[p]
        red_ref[...] = red
        out_ref[pl.ds(me * ch, ch), :] = red

        p2_sends = []
        for q in range(N_DEV):
            rdma = pltpu.make_async_remote_copy(
                src_ref=red_ref,
                dst_ref=out_ref.at[pl.ds(me * ch, ch), :],
                send_sem=send2.at[q],
                recv_sem=recv2.at[me],
                device_id=(q,),
                device_id_type=pl.DeviceIdType.MESH,
            )
            p2_sends.append(rdma)

            @pl.when(me != q)
            def _(rdma=rdma):
                rdma.start()

        for p in range(N_DEV):
            recv = pltpu.make_async_remote_copy(
                src_ref=red_ref,
                dst_ref=out_ref.at[pl.ds(p * ch, ch), :],
                send_sem=send2.at[p],
                recv_sem=recv2.at[p],
                device_id=(p,),
                device_id_type=pl.DeviceIdType.MESH,
            )

            @pl.when(me != p)
            def _(recv=recv):
                recv.wait_recv()

        for q in range(N_DEV):
            @pl.when(me != q)
            def _(s1=p1_sends[q], s2=p2_sends[q]):
                s1.wait_send()
                s2.wait_send()

    out_shape = jax.ShapeDtypeStruct((m, n), jnp.bfloat16)
    return pl.pallas_call(
        body,
        out_shape=out_shape,
        in_specs=[pl.BlockSpec(memory_space=pltpu.VMEM)],
        out_specs=pl.BlockSpec(memory_space=pltpu.VMEM),
        scratch_shapes=[
            pltpu.VMEM((m, n), jnp.bfloat16),
            pltpu.VMEM((N_DEV, ch, n), jnp.bfloat16),
            pltpu.VMEM((ch, n), jnp.bfloat16),
            pltpu.SemaphoreType.DMA((N_DEV,)),
            pltpu.SemaphoreType.DMA((N_DEV,)),
            pltpu.SemaphoreType.DMA((N_DEV,)),
            pltpu.SemaphoreType.DMA((N_DEV,)),
        ],
    )(x)


# device time: 14991 ns/iter; 1.2659x vs baseline; 1.2659x over previous
import jax
import jax.numpy as jnp
from jax import lax
from jax.experimental import pallas as pl
from jax.experimental.pallas import tpu as pltpu

N_DEV = 8
OFFSETS = (6, 2, 5, 7, 1, 3, 4)


def kernel(x):
    _, m, n = x.shape
    ch = m // N_DEV

    def body(x_ref, out_ref, xb_ref, comm_ref, send1, recv1, send2, recv2):
        me = lax.axis_index("i")

        xb_ref[...] = x_ref[0].astype(jnp.bfloat16)

        barrier_sem = pltpu.get_barrier_semaphore()
        for d in OFFSETS:
            pl.semaphore_signal(
                barrier_sem, inc=1,
                device_id=(me ^ d,), device_id_type=pl.DeviceIdType.MESH,
            )
        pl.semaphore_wait(barrier_sem, N_DEV - 1)

        p1_sends = []
        for d in OFFSETS:
            q = me ^ d
            rdma = pltpu.make_async_remote_copy(
                src_ref=xb_ref.at[pl.ds(q * ch, ch), :],
                dst_ref=comm_ref.at[me],
                send_sem=send1.at[d],
                recv_sem=recv1.at[me],
                device_id=(q,),
                device_id_type=pl.DeviceIdType.MESH,
            )
            rdma.start()
            p1_sends.append(rdma)

        comm_ref[pl.ds(me, 1)] = xb_ref[pl.ds(me * ch, ch), :][None]

        for d in OFFSETS:
            p = me ^ d
            recv = pltpu.make_async_remote_copy(
                src_ref=comm_ref.at[p],
                dst_ref=comm_ref.at[p],
                send_sem=send1.at[d],
                recv_sem=recv1.at[p],
                device_id=(p,),
                device_id_type=pl.DeviceIdType.MESH,
            )
            recv.wait_recv()

        red = comm_ref[0]
        for p in range(1, N_DEV):
            red = red + comm_ref[p]
        out_ref[pl.ds(me * ch, ch), :] = red

        p2_sends = []
        for d in OFFSETS:
            q = me ^ d
            rdma = pltpu.make_async_remote_copy(
                src_ref=out_ref.at[pl.ds(me * ch, ch), :],
                dst_ref=out_ref.at[pl.ds(me * ch, ch), :],
                send_sem=send2.at[d],
                recv_sem=recv2.at[me],
                device_id=(q,),
                device_id_type=pl.DeviceIdType.MESH,
            )
            rdma.start()
            p2_sends.append(rdma)

        for d in OFFSETS:
            p = me ^ d
            recv = pltpu.make_async_remote_copy(
                src_ref=out_ref.at[pl.ds(p * ch, ch), :],
                dst_ref=out_ref.at[pl.ds(p * ch, ch), :],
                send_sem=send2.at[d],
                recv_sem=recv2.at[p],
                device_id=(p,),
                device_id_type=pl.DeviceIdType.MESH,
            )
            recv.wait_recv()

        for s in p1_sends:
            s.wait_send()
        for s in p2_sends:
            s.wait_send()

    out_shape = jax.ShapeDtypeStruct((m, n), jnp.bfloat16)
    return pl.pallas_call(
        body,
        out_shape=out_shape,
        in_specs=[pl.BlockSpec(memory_space=pltpu.VMEM)],
        out_specs=pl.BlockSpec(memory_space=pltpu.VMEM),
        scratch_shapes=[
            pltpu.VMEM((m, n), jnp.bfloat16),
            pltpu.VMEM((N_DEV, ch, n), jnp.bfloat16),
            pltpu.SemaphoreType.DMA((N_DEV,)),
            pltpu.SemaphoreType.DMA((N_DEV,)),
            pltpu.SemaphoreType.DMA((N_DEV,)),
            pltpu.SemaphoreType.DMA((N_DEV,)),
        ],
        compiler_params=pltpu.CompilerParams(collective_id=0),
    )(x)
